# baseline (speedup 1.0000x reference)
import functools

import jax
import jax.numpy as jnp
from jax import lax
from jax.experimental import pallas as pl
from jax.experimental.pallas import tpu as pltpu
from jax.experimental.pallas import tpu_sc as plsc

N = 10000
E = 320000
D = 128
K = 8
NC = 2
NS = 16
NW = NC * NS
CHUNK = 128
EPT = -(-E // NW)
CH = -(-EPT // CHUNK)
EPTP = CH * CHUNK
NP = 10240
SL = NP // NS
RB = 512
GRID = NP // RB


def _w_id():
    c = lax.axis_index("c")
    s = lax.axis_index("s")
    return c, s, c * NS + s


_DEG_SCRATCH = [
    pltpu.VMEM((CH, CHUNK), jnp.int32),
    pltpu.VMEM((CHUNK, 16), jnp.float32),
    pltpu.VMEM((SL, 16), jnp.float32),
    pltpu.VMEM((SL // CHUNK, CHUNK), jnp.int32),
    pltpu.VMEM_SHARED((NP, 16), jnp.float32),
    pltpu.SemaphoreType.DMA,
]
_DEG_OUT = jax.ShapeDtypeStruct((NC, NP, 16), jnp.float32)


def _deg_body(srcp_hbm, deg_hbm, idx_v, ones_v, zero_v, id_v, acc_sh, sem):
        c, s, w = _w_id()
        base = s * SL

        def fill_ones(i, _):
            ones_v[i, :] = jnp.ones((16,), jnp.float32)
            return 0

        lax.fori_loop(0, CHUNK, fill_ones, 0)

        def fill_zero(i, _):
            zero_v[i, :] = jnp.zeros((16,), jnp.float32)
            return 0

        lax.fori_loop(0, SL, fill_zero, 0)

        def fill_id(i, _):
            def fill_g(g, _2):
                id_v[i, pl.ds(g * 16, 16)] = (
                    base + i * CHUNK + g * 16 + lax.iota(jnp.int32, 16)
                )
                return 0

            lax.fori_loop(0, CHUNK // 16, fill_g, 0)
            return 0

        lax.fori_loop(0, SL // CHUNK, fill_id, 0)

        for r in range(SL // CHUNK):
            pltpu.sync_copy(
                zero_v.at[pl.ds(r * CHUNK, CHUNK)], acc_sh.at[id_v.at[r]]
            )
        pltpu.sync_copy(srcp_hbm.at[w], idx_v)
        plsc.subcore_barrier()

        def body(j, _):
            pltpu.sync_copy(ones_v, acc_sh.at[idx_v.at[j]], add=True)
            return 0

        lax.fori_loop(0, CH, body, 0)
        plsc.subcore_barrier()
        for r in range(SL // CHUNK):
            pltpu.async_copy(
                acc_sh.at[id_v.at[r]], zero_v.at[pl.ds(r * CHUNK, CHUNK)], sem
            ).wait()
        pltpu.sync_copy(zero_v, deg_hbm.at[c, pl.ds(s * SL, SL)])


NBUF = 2
CHH = -(-CH // 2)

_PROP_SCRATCH = [
    pltpu.VMEM((CHH, CHUNK), jnp.int32),
    pltpu.VMEM((CHH, CHUNK), jnp.int32),
    pltpu.VMEM((NBUF, CHUNK, D), jnp.float32),
    pltpu.VMEM((8, D), jnp.float32),
    pltpu.VMEM_SHARED((NP, D), jnp.float32),
    pltpu.SemaphoreType.DMA((NBUF,)),
    pltpu.SemaphoreType.DMA((NBUF,)),
]
_PROP_OUT = jax.ShapeDtypeStruct((NC, NP, D), jnp.float32)


def _prop_body(y_hbm, srcp_hbm, dst_hbm, s_hbm, src_v, dst_v, rows_v,
               zero_v, acc_sh, sem, sem_s):
        c, s, w = _w_id()

        def fill_zero(i, _):
            for g in range(D // 16):
                zero_v[i, pl.ds(g * 16, 16)] = jnp.zeros((16,), jnp.float32)
            return 0

        lax.fori_loop(0, 8, fill_zero, 0)
        for r in range(SL // 8):
            pltpu.sync_copy(zero_v, acc_sh.at[pl.ds(s * SL + r * 8, 8)])
        plsc.subcore_barrier()

        def sg(j):
            p = lax.rem(j, NBUF)
            pltpu.async_copy(y_hbm.at[src_v.at[j]], rows_v.at[p], sem.at[p])

        def wg(j):
            p = lax.rem(j, NBUF)
            pltpu.make_async_copy(
                y_hbm.at[src_v.at[j]], rows_v.at[p], sem.at[p]
            ).wait()

        def ss(j):
            p = lax.rem(j, NBUF)
            pltpu.async_copy(
                rows_v.at[p], acc_sh.at[dst_v.at[j]], sem_s.at[p], add=True
            )

        def ws(j):
            p = lax.rem(j, NBUF)
            pltpu.make_async_copy(
                rows_v.at[p], acc_sh.at[dst_v.at[j]], sem_s.at[p]
            ).wait()

        for h in range(2):
            lo = h * CHH
            hc = min(CH - lo, CHH)
            pltpu.sync_copy(
                srcp_hbm.at[w, pl.ds(lo, hc)], src_v.at[pl.ds(0, hc)]
            )
            pltpu.sync_copy(
                dst_hbm.at[w, pl.ds(lo, hc)], dst_v.at[pl.ds(0, hc)]
            )
            j0 = jnp.int32(0)
            sg(j0)
            wg(j0)
            ss(j0)
            sg(jnp.int32(1))

            def body(j, _):
                wg(j)
                ss(j)
                ws(j - 1)
                sg(j + 1)
                return 0

            lax.fori_loop(1, hc - 1, body, 0)
            jl = jnp.int32(hc - 1)
            wg(jl)
            ss(jl)
            ws(jl - 1)
            ws(jl)
        plsc.subcore_barrier()
        pltpu.sync_copy(
            acc_sh.at[pl.ds(s * SL, SL)], s_hbm.at[c, pl.ds(s * SL, SL)]
        )


@functools.cache
def _build_sc_kernels():
    mesh = plsc.VectorSubcoreMesh(
        core_axis_name="c", subcore_axis_name="s", num_cores=NC, num_subcores=NS
    )
    prop_kernel = pl.kernel(
        _prop_body, out_type=_PROP_OUT, mesh=mesh, scratch_types=_PROP_SCRATCH
    )
    return prop_kernel


def _dinv_body(degp_ref, dinv_ref):
    i = pl.program_id(0)
    deg = degp_ref[0, :, 0] + degp_ref[1, :, 0]
    dinv = jnp.where(deg > 0, lax.rsqrt(jnp.maximum(deg, 1e-20)), 0.0)
    rid = lax.broadcasted_iota(jnp.int32, (RB,), 0) + i * RB
    dinv = jnp.where(rid < N, dinv, 0.0)
    dinv_ref[...] = dinv[:, None]


def _scale_body(x_ref, dinv_ref, y_ref):
    y_ref[...] = x_ref[...] * dinv_ref[...]


def _make_rec_body(a, e):
    def body(sp_ref, dinv_ref, prev_ref, tx_ref, y_ref):
        sblk = sp_ref[0] + sp_ref[1]
        dv = dinv_ref[...]
        tx = a * (dv * sblk) + e * prev_ref[...]
        tx_ref[...] = tx
        y_ref[...] = dv * tx

    return body


def _matmul_body(*refs):
    tx_refs = refs[:K]
    w_ref, b_ref, out_ref = refs[K], refs[K + 1], refs[K + 2]
    acc = jnp.zeros((RB, D), jnp.float32) + b_ref[...]
    for k in range(K):
        acc = acc + jnp.dot(
            tx_refs[k][...], w_ref[k], preferred_element_type=jnp.float32
        )
    out_ref[...] = jnp.maximum(acc, 0.0)


def _row_blocks(nd=D):
    return pl.BlockSpec((RB, nd), lambda i: (i, 0))


def kernel(x, edge_index, W, b):
    prop_kernel = _build_sc_kernels()

    src = edge_index[0].astype(jnp.int32)
    dst = edge_index[1].astype(jnp.int32)
    srcp = jnp.where(src == dst, N, src)
    pad = EPTP * NW - E
    srcp = jnp.concatenate([srcp, jnp.full((pad,), N, jnp.int32)]).reshape(
        NW, EPTP
    )
    dstp = jnp.concatenate([dst, jnp.full((pad,), N, jnp.int32)]).reshape(
        NW, EPTP
    )
    order = jnp.argsort(dstp, axis=1)
    srcp = jnp.take_along_axis(srcp, order, axis=1).reshape(NW, CH, CHUNK)
    dstp = jnp.take_along_axis(dstp, order, axis=1).reshape(NW, CH, CHUNK)
    x_pad = jnp.concatenate([x, jnp.zeros((NP - N, D), jnp.float32)], axis=0)

    ones_table = jnp.ones((NP, D), jnp.float32)
    deg_p = prop_kernel(ones_table, dstp, srcp)

    dinv = pl.pallas_call(
        _dinv_body,
        grid=(GRID,),
        in_specs=[pl.BlockSpec((NC, RB, D), lambda i: (0, i, 0))],
        out_specs=_row_blocks(1),
        out_shape=jax.ShapeDtypeStruct((NP, 1), jnp.float32),
    )(deg_p)

    y = pl.pallas_call(
        _scale_body,
        grid=(GRID,),
        in_specs=[_row_blocks(), _row_blocks(1)],
        out_specs=_row_blocks(),
        out_shape=jax.ShapeDtypeStruct((NP, D), jnp.float32),
    )(x_pad, dinv)

    txs = [x_pad]
    prev = x_pad
    for k in range(1, K):
        s_p = prop_kernel(y, srcp, dstp)
        a, e = (-1.0, 0.0) if k == 1 else (-2.0, -1.0)
        tx, y = pl.pallas_call(
            _make_rec_body(a, e),
            grid=(GRID,),
            in_specs=[
                pl.BlockSpec((NC, RB, D), lambda i: (0, i, 0)),
                _row_blocks(1),
                _row_blocks(),
            ],
            out_specs=[_row_blocks(), _row_blocks()],
            out_shape=[
                jax.ShapeDtypeStruct((NP, D), jnp.float32),
                jax.ShapeDtypeStruct((NP, D), jnp.float32),
            ],
        )(s_p, dinv, prev)
        prev = txs[-1]
        txs.append(tx)

    out = pl.pallas_call(
        _matmul_body,
        grid=(GRID,),
        in_specs=[_row_blocks() for _ in range(K)]
        + [
            pl.BlockSpec((K, D, D), lambda i: (0, 0, 0)),
            pl.BlockSpec((1, D), lambda i: (0, 0)),
        ],
        out_specs=_row_blocks(),
        out_shape=jax.ShapeDtypeStruct((N, D), jnp.float32),
    )(*txs, W, b.reshape(1, D))
    return out


# speedup vs baseline: 1.0619x; 1.0619x over previous
import functools

import jax
import jax.numpy as jnp
from jax import lax
from jax.experimental import pallas as pl
from jax.experimental.pallas import tpu as pltpu
from jax.experimental.pallas import tpu_sc as plsc

N = 10000
E = 320000
D = 128
K = 8
NC = 2
NS = 16
NW = NC * NS
CHUNK = 128
EPT = -(-E // NW)
CH = -(-EPT // CHUNK)
EPTP = CH * CHUNK
NP = 10240
SL = NP // NS
RB = 512
GRID = NP // RB


def _w_id():
    c = lax.axis_index("c")
    s = lax.axis_index("s")
    return c, s, c * NS + s


_DEG_SCRATCH = [
    pltpu.VMEM((CH, CHUNK), jnp.int32),
    pltpu.VMEM((CHUNK, 16), jnp.float32),
    pltpu.VMEM((SL, 16), jnp.float32),
    pltpu.VMEM((SL // CHUNK, CHUNK), jnp.int32),
    pltpu.VMEM_SHARED((NP, 16), jnp.float32),
    pltpu.SemaphoreType.DMA,
]
_DEG_OUT = jax.ShapeDtypeStruct((NC, NP, 16), jnp.float32)


def _deg_body(srcp_hbm, deg_hbm, idx_v, ones_v, zero_v, id_v, acc_sh, sem):
        c, s, w = _w_id()
        base = s * SL

        def fill_ones(i, _):
            ones_v[i, :] = jnp.ones((16,), jnp.float32)
            return 0

        lax.fori_loop(0, CHUNK, fill_ones, 0)

        def fill_zero(i, _):
            zero_v[i, :] = jnp.zeros((16,), jnp.float32)
            return 0

        lax.fori_loop(0, SL, fill_zero, 0)

        def fill_id(i, _):
            def fill_g(g, _2):
                id_v[i, pl.ds(g * 16, 16)] = (
                    base + i * CHUNK + g * 16 + lax.iota(jnp.int32, 16)
                )
                return 0

            lax.fori_loop(0, CHUNK // 16, fill_g, 0)
            return 0

        lax.fori_loop(0, SL // CHUNK, fill_id, 0)

        for r in range(SL // CHUNK):
            pltpu.sync_copy(
                zero_v.at[pl.ds(r * CHUNK, CHUNK)], acc_sh.at[id_v.at[r]]
            )
        pltpu.sync_copy(srcp_hbm.at[w], idx_v)
        plsc.subcore_barrier()

        def body(j, _):
            pltpu.sync_copy(ones_v, acc_sh.at[idx_v.at[j]], add=True)
            return 0

        lax.fori_loop(0, CH, body, 0)
        plsc.subcore_barrier()
        for r in range(SL // CHUNK):
            pltpu.async_copy(
                acc_sh.at[id_v.at[r]], zero_v.at[pl.ds(r * CHUNK, CHUNK)], sem
            ).wait()
        pltpu.sync_copy(zero_v, deg_hbm.at[c, pl.ds(s * SL, SL)])


NBUF = 2
CHH = -(-CH // 2)

_PROP_SCRATCH = [
    pltpu.VMEM((CHH, CHUNK), jnp.int32),
    pltpu.VMEM((CHH, CHUNK), jnp.int32),
    pltpu.VMEM((NBUF, CHUNK, D), jnp.float32),
    pltpu.VMEM((8, D), jnp.float32),
    pltpu.VMEM_SHARED((NP, D), jnp.float32),
    pltpu.SemaphoreType.DMA((NBUF,)),
    pltpu.SemaphoreType.DMA((NBUF,)),
]
_PROP_OUT = jax.ShapeDtypeStruct((NC, NP, D), jnp.float32)


def _prop_body(y_hbm, srcp_hbm, dst_hbm, s_hbm, src_v, dst_v, rows_v,
               zero_v, acc_sh, sem, sem_s):
        c, s, w = _w_id()

        def fill_zero(i, _):
            for g in range(D // 16):
                zero_v[i, pl.ds(g * 16, 16)] = jnp.zeros((16,), jnp.float32)
            return 0

        lax.fori_loop(0, 8, fill_zero, 0)
        for r in range(SL // 8):
            pltpu.sync_copy(zero_v, acc_sh.at[pl.ds(s * SL + r * 8, 8)])
        plsc.subcore_barrier()

        def sg(j):
            p = lax.rem(j, NBUF)
            pltpu.async_copy(y_hbm.at[src_v.at[j]], rows_v.at[p], sem.at[p])

        def wg(j):
            p = lax.rem(j, NBUF)
            pltpu.make_async_copy(
                y_hbm.at[src_v.at[j]], rows_v.at[p], sem.at[p]
            ).wait()

        for h in range(2):
            lo = h * CHH
            hc = min(CH - lo, CHH)
            pltpu.sync_copy(
                srcp_hbm.at[w, pl.ds(lo, hc)], src_v.at[pl.ds(0, hc)]
            )
            pltpu.sync_copy(
                dst_hbm.at[w, pl.ds(lo, hc)], dst_v.at[pl.ds(0, hc)]
            )
            def finish(j):
                wg(j)
                p = lax.rem(j, NBUF)
                pltpu.sync_copy(
                    rows_v.at[p], acc_sh.at[dst_v.at[j]], add=True
                )

            for j0 in range(NBUF - 1):
                sg(jnp.int32(j0))

            def body(j, _):
                sg(j + (NBUF - 1))
                finish(j)
                return 0

            lax.fori_loop(0, hc - (NBUF - 1), body, 0)
            for j0 in range(hc - (NBUF - 1), hc):
                finish(jnp.int32(j0))
        plsc.subcore_barrier()
        pltpu.sync_copy(
            acc_sh.at[pl.ds(s * SL, SL)], s_hbm.at[c, pl.ds(s * SL, SL)]
        )


@functools.cache
def _build_sc_kernels():
    mesh = plsc.VectorSubcoreMesh(
        core_axis_name="c", subcore_axis_name="s", num_cores=NC, num_subcores=NS
    )
    prop_kernel = pl.kernel(
        _prop_body, out_type=_PROP_OUT, mesh=mesh, scratch_types=_PROP_SCRATCH
    )
    return prop_kernel


def _dinv_body(degp_ref, dinv_ref):
    i = pl.program_id(0)
    deg = degp_ref[0, :, 0] + degp_ref[1, :, 0]
    dinv = jnp.where(deg > 0, lax.rsqrt(jnp.maximum(deg, 1e-20)), 0.0)
    rid = lax.broadcasted_iota(jnp.int32, (RB,), 0) + i * RB
    dinv = jnp.where(rid < N, dinv, 0.0)
    dinv_ref[...] = dinv[:, None]


def _scale_body(x_ref, dinv_ref, y_ref):
    y_ref[...] = x_ref[...] * dinv_ref[...]


def _make_rec_body(a, e):
    def body(sp_ref, dinv_ref, prev_ref, tx_ref, y_ref):
        sblk = sp_ref[0] + sp_ref[1]
        dv = dinv_ref[...]
        tx = a * (dv * sblk) + e * prev_ref[...]
        tx_ref[...] = tx
        y_ref[...] = dv * tx

    return body


def _matmul_body(*refs):
    tx_refs = refs[:K]
    w_ref, b_ref, out_ref = refs[K], refs[K + 1], refs[K + 2]
    acc = jnp.zeros((RB, D), jnp.float32) + b_ref[...]
    for k in range(K):
        acc = acc + jnp.dot(
            tx_refs[k][...], w_ref[k], preferred_element_type=jnp.float32
        )
    out_ref[...] = jnp.maximum(acc, 0.0)


def _row_blocks(nd=D):
    return pl.BlockSpec((RB, nd), lambda i: (i, 0))


def kernel(x, edge_index, W, b):
    prop_kernel = _build_sc_kernels()

    src = edge_index[0].astype(jnp.int32)
    dst = edge_index[1].astype(jnp.int32)
    srcp = jnp.where(src == dst, N, src)
    pad = EPTP * NW - E
    srcp = jnp.concatenate([srcp, jnp.full((pad,), N, jnp.int32)]).reshape(
        NW, CH, CHUNK
    )
    dstp = jnp.concatenate([dst, jnp.full((pad,), N, jnp.int32)]).reshape(
        NW, CH, CHUNK
    )
    x_pad = jnp.concatenate([x, jnp.zeros((NP - N, D), jnp.float32)], axis=0)

    ones_table = jnp.ones((NP, D), jnp.float32)
    deg_p = prop_kernel(ones_table, dstp, srcp)

    dinv = pl.pallas_call(
        _dinv_body,
        grid=(GRID,),
        in_specs=[pl.BlockSpec((NC, RB, D), lambda i: (0, i, 0))],
        out_specs=_row_blocks(1),
        out_shape=jax.ShapeDtypeStruct((NP, 1), jnp.float32),
    )(deg_p)

    y = pl.pallas_call(
        _scale_body,
        grid=(GRID,),
        in_specs=[_row_blocks(), _row_blocks(1)],
        out_specs=_row_blocks(),
        out_shape=jax.ShapeDtypeStruct((NP, D), jnp.float32),
    )(x_pad, dinv)

    txs = [x_pad]
    prev = x_pad
    for k in range(1, K):
        s_p = prop_kernel(y, srcp, dstp)
        a, e = (-1.0, 0.0) if k == 1 else (-2.0, -1.0)
        tx, y = pl.pallas_call(
            _make_rec_body(a, e),
            grid=(GRID,),
            in_specs=[
                pl.BlockSpec((NC, RB, D), lambda i: (0, i, 0)),
                _row_blocks(1),
                _row_blocks(),
            ],
            out_specs=[_row_blocks(), _row_blocks()],
            out_shape=[
                jax.ShapeDtypeStruct((NP, D), jnp.float32),
                jax.ShapeDtypeStruct((NP, D), jnp.float32),
            ],
        )(s_p, dinv, prev)
        prev = txs[-1]
        txs.append(tx)

    out = pl.pallas_call(
        _matmul_body,
        grid=(GRID,),
        in_specs=[_row_blocks() for _ in range(K)]
        + [
            pl.BlockSpec((K, D, D), lambda i: (0, 0, 0)),
            pl.BlockSpec((1, D), lambda i: (0, 0)),
        ],
        out_specs=_row_blocks(),
        out_shape=jax.ShapeDtypeStruct((N, D), jnp.float32),
    )(*txs, W, b.reshape(1, D))
    return out


# speedup vs baseline: 2.5292x; 2.3819x over previous
import functools

import jax
import jax.numpy as jnp
from jax import lax
from jax.experimental import pallas as pl
from jax.experimental.pallas import tpu as pltpu
from jax.experimental.pallas import tpu_sc as plsc

N = 10000
E = 320000
D = 128
K = 8
NC = 2
NS = 16
NW = NC * NS
CHUNK = 128
EPT = -(-E // NW)
CH = -(-EPT // CHUNK)
EPTP = CH * CHUNK
NP = 10240
SL = NP // NS
RB = 512
GRID = NP // RB


def _w_id():
    c = lax.axis_index("c")
    s = lax.axis_index("s")
    return c, s, c * NS + s


_DEG_SCRATCH = [
    pltpu.VMEM((CH, CHUNK), jnp.int32),
    pltpu.VMEM((CHUNK, 16), jnp.float32),
    pltpu.VMEM((SL, 16), jnp.float32),
    pltpu.VMEM((SL // CHUNK, CHUNK), jnp.int32),
    pltpu.VMEM_SHARED((NP, 16), jnp.float32),
    pltpu.SemaphoreType.DMA,
]
_DEG_OUT = jax.ShapeDtypeStruct((NC, NP, 16), jnp.float32)


def _deg_body(srcp_hbm, deg_hbm, idx_v, ones_v, zero_v, id_v, acc_sh, sem):
        c, s, w = _w_id()
        base = s * SL

        def fill_ones(i, _):
            ones_v[i, :] = jnp.ones((16,), jnp.float32)
            return 0

        lax.fori_loop(0, CHUNK, fill_ones, 0)

        def fill_zero(i, _):
            zero_v[i, :] = jnp.zeros((16,), jnp.float32)
            return 0

        lax.fori_loop(0, SL, fill_zero, 0)

        def fill_id(i, _):
            def fill_g(g, _2):
                id_v[i, pl.ds(g * 16, 16)] = (
                    base + i * CHUNK + g * 16 + lax.iota(jnp.int32, 16)
                )
                return 0

            lax.fori_loop(0, CHUNK // 16, fill_g, 0)
            return 0

        lax.fori_loop(0, SL // CHUNK, fill_id, 0)

        for r in range(SL // CHUNK):
            pltpu.sync_copy(
                zero_v.at[pl.ds(r * CHUNK, CHUNK)], acc_sh.at[id_v.at[r]]
            )
        pltpu.sync_copy(srcp_hbm.at[w], idx_v)
        plsc.subcore_barrier()

        def body(j, _):
            pltpu.sync_copy(ones_v, acc_sh.at[idx_v.at[j]], add=True)
            return 0

        lax.fori_loop(0, CH, body, 0)
        plsc.subcore_barrier()
        for r in range(SL // CHUNK):
            pltpu.async_copy(
                acc_sh.at[id_v.at[r]], zero_v.at[pl.ds(r * CHUNK, CHUNK)], sem
            ).wait()
        pltpu.sync_copy(zero_v, deg_hbm.at[c, pl.ds(s * SL, SL)])


NBUF = 2
CHH = -(-CH // 2)

_PROP_SCRATCH = [
    pltpu.VMEM((CHH, CHUNK), jnp.int32),
    pltpu.VMEM((CHH, CHUNK), jnp.int32),
    pltpu.VMEM((NBUF, CHUNK, D), jnp.float32),
    pltpu.VMEM((8, D), jnp.float32),
    pltpu.VMEM_SHARED((NP, D), jnp.float32),
    pltpu.SemaphoreType.DMA((NBUF,)),
    pltpu.SemaphoreType.DMA((NBUF,)),
]
_PROP_OUT = jax.ShapeDtypeStruct((NC, NP, D), jnp.float32)


def _prop_body(y_hbm, srcp_hbm, dst_hbm, s_hbm, src_v, dst_v, rows_v,
               zero_v, acc_sh, sem, sem_s):
        c, s, w = _w_id()

        def fill_zero(i, _):
            for g in range(D // 16):
                zero_v[i, pl.ds(g * 16, 16)] = jnp.zeros((16,), jnp.float32)
            return 0

        lax.fori_loop(0, 8, fill_zero, 0)
        for r in range(SL // 8):
            pltpu.sync_copy(zero_v, acc_sh.at[pl.ds(s * SL + r * 8, 8)])
        plsc.subcore_barrier()

        def sg(j):
            p = lax.rem(j, NBUF)
            pltpu.async_copy(y_hbm.at[src_v.at[j]], rows_v.at[p], sem.at[p])

        def wg(j):
            p = lax.rem(j, NBUF)
            pltpu.make_async_copy(
                y_hbm.at[src_v.at[j]], rows_v.at[p], sem.at[p]
            ).wait()

        for h in range(2):
            lo = h * CHH
            hc = min(CH - lo, CHH)
            pltpu.sync_copy(
                srcp_hbm.at[w, pl.ds(lo, hc)], src_v.at[pl.ds(0, hc)]
            )
            pltpu.sync_copy(
                dst_hbm.at[w, pl.ds(lo, hc)], dst_v.at[pl.ds(0, hc)]
            )
            def finish(j):
                wg(j)
                p = lax.rem(j, NBUF)
                pltpu.sync_copy(
                    rows_v.at[p], acc_sh.at[dst_v.at[j]], add=True
                )

            for j0 in range(NBUF - 1):
                sg(jnp.int32(j0))

            def body(j, _):
                sg(j + (NBUF - 1))
                finish(j)
                return 0

            lax.fori_loop(0, hc - (NBUF - 1), body, 0)
            for j0 in range(hc - (NBUF - 1), hc):
                finish(jnp.int32(j0))
        plsc.subcore_barrier()
        pltpu.sync_copy(
            acc_sh.at[pl.ds(s * SL, SL)], s_hbm.at[c, pl.ds(s * SL, SL)]
        )


@functools.cache
def _build_sc_kernels():
    mesh = plsc.VectorSubcoreMesh(
        core_axis_name="c", subcore_axis_name="s", num_cores=NC, num_subcores=NS
    )
    prop_kernel = pl.kernel(
        _prop_body, out_type=_PROP_OUT, mesh=mesh, scratch_types=_PROP_SCRATCH
    )
    return prop_kernel


def _dinv_body(degp_ref, dinv_ref):
    i = pl.program_id(0)
    deg = degp_ref[0, :, 0] + degp_ref[1, :, 0]
    dinv = jnp.where(deg > 0, lax.rsqrt(jnp.maximum(deg, 1e-20)), 0.0)
    rid = lax.broadcasted_iota(jnp.int32, (RB,), 0) + i * RB
    dinv = jnp.where(rid < N, dinv, 0.0)
    dinv_ref[...] = dinv[:, None]


def _scale_body(x_ref, dinv_ref, y_ref):
    y_ref[...] = x_ref[...] * dinv_ref[...]


def _make_rec_body(a, e):
    def body(sp_ref, dinv_ref, prev_ref, tx_ref, y_ref):
        sblk = sp_ref[0] + sp_ref[1]
        dv = dinv_ref[...]
        tx = a * (dv * sblk) + e * prev_ref[...]
        tx_ref[...] = tx
        y_ref[...] = dv * tx

    return body


def _matmul_body(*refs):
    tx_refs = refs[:K]
    w_ref, b_ref, out_ref = refs[K], refs[K + 1], refs[K + 2]
    acc = jnp.zeros((RB, D), jnp.float32) + b_ref[...]
    for k in range(K):
        acc = acc + jnp.dot(
            tx_refs[k][...], w_ref[k], preferred_element_type=jnp.float32
        )
    out_ref[...] = jnp.maximum(acc, 0.0)


def _row_blocks(nd=D):
    return pl.BlockSpec((RB, nd), lambda i: (i, 0))


def kernel(x, edge_index, W, b):
    prop_kernel = _build_sc_kernels()

    src = edge_index[0].astype(jnp.int32)
    dst = edge_index[1].astype(jnp.int32)
    srcp = jnp.where(src == dst, N, src)
    pad = EPTP * NW - E
    spread = N + (jnp.arange(pad, dtype=jnp.int32) % (NP - N))
    srcp = jnp.concatenate([srcp, spread]).reshape(NW, CH, CHUNK)
    dstp = jnp.concatenate([dst, spread]).reshape(NW, CH, CHUNK)
    x_pad = jnp.concatenate([x, jnp.zeros((NP - N, D), jnp.float32)], axis=0)

    ones_table = jnp.ones((NP, D), jnp.float32)
    deg_p = prop_kernel(ones_table, dstp, srcp)

    dinv = pl.pallas_call(
        _dinv_body,
        grid=(GRID,),
        in_specs=[pl.BlockSpec((NC, RB, D), lambda i: (0, i, 0))],
        out_specs=_row_blocks(1),
        out_shape=jax.ShapeDtypeStruct((NP, 1), jnp.float32),
    )(deg_p)

    y = pl.pallas_call(
        _scale_body,
        grid=(GRID,),
        in_specs=[_row_blocks(), _row_blocks(1)],
        out_specs=_row_blocks(),
        out_shape=jax.ShapeDtypeStruct((NP, D), jnp.float32),
    )(x_pad, dinv)

    txs = [x_pad]
    prev = x_pad
    for k in range(1, K):
        s_p = prop_kernel(y, srcp, dstp)
        a, e = (-1.0, 0.0) if k == 1 else (-2.0, -1.0)
        tx, y = pl.pallas_call(
            _make_rec_body(a, e),
            grid=(GRID,),
            in_specs=[
                pl.BlockSpec((NC, RB, D), lambda i: (0, i, 0)),
                _row_blocks(1),
                _row_blocks(),
            ],
            out_specs=[_row_blocks(), _row_blocks()],
            out_shape=[
                jax.ShapeDtypeStruct((NP, D), jnp.float32),
                jax.ShapeDtypeStruct((NP, D), jnp.float32),
            ],
        )(s_p, dinv, prev)
        prev = txs[-1]
        txs.append(tx)

    out = pl.pallas_call(
        _matmul_body,
        grid=(GRID,),
        in_specs=[_row_blocks() for _ in range(K)]
        + [
            pl.BlockSpec((K, D, D), lambda i: (0, 0, 0)),
            pl.BlockSpec((1, D), lambda i: (0, 0)),
        ],
        out_specs=_row_blocks(),
        out_shape=jax.ShapeDtypeStruct((N, D), jnp.float32),
    )(*txs, W, b.reshape(1, D))
    return out
